# baseline (speedup 1.0000x reference)
import functools

import jax
import jax.numpy as jnp
from jax.experimental import pallas as pl
from jax.experimental.pallas import tpu as pltpu

N = 10000
NP = 10240
E = 320000
EC = 256
NCHUNK = E // EC
G = 64
K = 64
F = 128
TL = 3 * F + 1
FP = 512
NSLOT = G * K
SB = 256
RB = 256
T2 = 32
TV = 28


def _gcn_first(ec_ref, er_ref, h_ref, w_ref, b_ref, out_ref, deg_ref,
               hlin, acc, dacc):
    c = pl.program_id(0)

    @pl.when(c == 0)
    def _init():
        hlin[...] = jax.lax.dot_general(
            h_ref[...], w_ref[...], (((1,), (1,)), ((), ())),
            preferred_element_type=jnp.float32) + b_ref[...]
        acc[...] = jnp.zeros((NP, F), jnp.float32)
        dacc[...] = jnp.zeros((NP, 1), jnp.float32)

    src_col = ec_ref[0, :, 0:1]
    src_row = er_ref[0, 0:1, :]
    dst_row = er_ref[0, 1:2, :]
    lane_n = jax.lax.broadcasted_iota(jnp.int32, (EC, NP), 1)
    oh_src = (src_col == lane_n).astype(jnp.float32)
    g = jnp.dot(oh_src, hlin[...], preferred_element_type=jnp.float32)
    node_col = jax.lax.broadcasted_iota(jnp.int32, (NP, EC), 0)
    oh_dst_t = (node_col == dst_row).astype(jnp.float32)
    acc[...] += jnp.dot(oh_dst_t, g, preferred_element_type=jnp.float32)
    oh_src_t = (node_col == src_row).astype(jnp.float32)
    dacc[...] += jnp.sum(oh_src_t, axis=1, keepdims=True)

    @pl.when(c == NCHUNK - 1)
    def _fin():
        deg = dacc[...] + 1.0
        out_ref[...] = jnp.tanh((acc[...] + hlin[...]) / deg)
        deg_ref[...] = deg


def _gcn_next(ec_ref, er_ref, h_ref, w_ref, b_ref, deg_ref, out_ref,
              hlin, acc):
    c = pl.program_id(0)

    @pl.when(c == 0)
    def _init():
        hlin[...] = jax.lax.dot_general(
            h_ref[...], w_ref[...], (((1,), (1,)), ((), ())),
            preferred_element_type=jnp.float32) + b_ref[...]
        acc[...] = jnp.zeros((NP, F), jnp.float32)

    src_col = ec_ref[0, :, 0:1]
    dst_row = er_ref[0, 1:2, :]
    lane_n = jax.lax.broadcasted_iota(jnp.int32, (EC, NP), 1)
    oh_src = (src_col == lane_n).astype(jnp.float32)
    g = jnp.dot(oh_src, hlin[...], preferred_element_type=jnp.float32)
    node_col = jax.lax.broadcasted_iota(jnp.int32, (NP, EC), 0)
    oh_dst_t = (node_col == dst_row).astype(jnp.float32)
    acc[...] += jnp.dot(oh_dst_t, g, preferred_element_type=jnp.float32)

    @pl.when(c == NCHUNK - 1)
    def _fin():
        out_ref[...] = jnp.tanh((acc[...] + hlin[...]) / deg_ref[...])


def _rank_kernel(kc_ref, bc_ref, kr_ref, br_ref, rank_ref, racc):
    j = pl.program_id(0)

    @pl.when(j == 0)
    def _init():
        racc[...] = jnp.zeros((1, NP), jnp.float32)

    kc = kc_ref[...]
    bc = bc_ref[...]
    kr = kr_ref[...]
    br = br_ref[...]
    jidx = j * RB + jax.lax.broadcasted_iota(jnp.int32, (RB, 1), 0)
    iidx = jax.lax.broadcasted_iota(jnp.int32, (RB, NP), 1)
    beats = (kc > kr) | ((kc == kr) & (jidx < iidx))
    mask = (bc == br) & beats
    racc[...] += jnp.sum(mask.astype(jnp.float32), axis=0, keepdims=True)

    @pl.when(j == NP // RB - 1)
    def _fin():
        rank_ref[...] = racc[...]


def _select_kernel(rr_ref, br_ref, h1_ref, h2_ref, h3_ref, h4_ref, s_ref):
    b = pl.program_id(0)
    s = b * SB + jax.lax.broadcasted_iota(jnp.int32, (SB, 1), 0)
    par = jax.lax.shift_right_logical(s, 11)
    within = jax.lax.bitwise_and(s, 2047)
    gidx = jax.lax.shift_right_logical(within, 5)
    tidx = jax.lax.bitwise_and(within, 31)
    r = 2 * tidx + par
    oh = ((br_ref[...] == gidx) &
          (rr_ref[...] == r.astype(jnp.float32))).astype(jnp.float32)
    s_ref[:, 0:F] = jnp.dot(oh, h1_ref[...], preferred_element_type=jnp.float32)
    s_ref[:, F:2 * F] = jnp.dot(oh, h2_ref[...], preferred_element_type=jnp.float32)
    s_ref[:, 2 * F:3 * F] = jnp.dot(oh, h3_ref[...], preferred_element_type=jnp.float32)
    s_ref[:, 3 * F:4 * F] = jnp.dot(oh, h4_ref[...], preferred_element_type=jnp.float32)


def _tail_kernel(s_ref, wc1_ref, cb1_ref, w2_ref, cb2_ref, wr_ref, db1_ref,
                 w2t_ref, db2_ref, out_ref):
    s_even = s_ref[0:G * T2, :]
    s_odd = s_ref[G * T2:NSLOT, :]
    wc1 = wc1_ref[...]
    r1e = jnp.maximum(jnp.dot(s_even, wc1, preferred_element_type=jnp.float32)
                      + cb1_ref[...], 0.0)
    r1o = jnp.maximum(jnp.dot(s_odd, wc1, preferred_element_type=jnp.float32)
                      + cb1_ref[...], 0.0)
    pooled = jnp.maximum(r1e, r1o)
    c2 = jnp.zeros((G * T2, 32), jnp.float32)
    for dt in range(5):
        if dt == 0:
            sh = pooled
        else:
            sh = jnp.concatenate(
                [pooled[dt:, :], jnp.zeros((dt, 16), jnp.float32)], axis=0)
        c2 = c2 + jnp.dot(sh, w2_ref[dt, :, :],
                          preferred_element_type=jnp.float32)
    c2 = jnp.maximum(c2 + cb2_ref[...], 0.0)
    lane_gt = jax.lax.broadcasted_iota(jnp.int32, (G, G * T2), 1)
    grow = jax.lax.broadcasted_iota(jnp.int32, (G, G * T2), 0)
    racc = jnp.zeros((G, 32), jnp.float32)
    for t in range(TV):
        sel = (lane_gt == grow * T2 + t).astype(jnp.float32)
        ct = jnp.dot(sel, c2, preferred_element_type=jnp.float32)
        racc = racc + jnp.dot(ct, wr_ref[t, :, :],
                              preferred_element_type=jnp.float32)
    r1d = jnp.maximum(racc + db1_ref[...], 0.0)
    logits = jnp.dot(r1d, w2t_ref[...],
                     preferred_element_type=jnp.float32) + db2_ref[...]
    m = jnp.max(logits, axis=1, keepdims=True)
    e = jnp.exp(logits - m)
    out_ref[...] = e / jnp.sum(e, axis=1, keepdims=True)


def kernel(x, edge_index, batch, emb, W0, b0, W1, b1, W2, b2, W3, b3,
           conv1_w, conv1_b, conv2_w, conv2_b, d1_w, d1_b, d2_w, d2_b):
    f32 = jnp.float32
    h0 = emb[x].astype(f32)
    h0 = jnp.pad(h0, ((0, NP - N), (0, 0)))
    ec = edge_index.T.reshape(NCHUNK, EC, 2)
    er = edge_index.reshape(2, NCHUNK, EC).transpose(1, 0, 2)
    w3p = jnp.pad(W3, ((0, F - W3.shape[0]), (0, 0)))
    b3p = jnp.pad(b3, (0, F - b3.shape[0]))
    batch_p = jnp.pad(batch, (0, NP - N), constant_values=G)
    batch_row = batch_p.reshape(1, NP)
    batch_col = batch_p.reshape(NP, 1)

    espec_c = pl.BlockSpec((1, EC, 2), lambda c: (c, 0, 0))
    espec_r = pl.BlockSpec((1, 2, EC), lambda c: (c, 0, 0))
    full_h = pl.BlockSpec((NP, F), lambda c: (0, 0))
    full_w = pl.BlockSpec((F, F), lambda c: (0, 0))
    full_b = pl.BlockSpec((1, F), lambda c: (0, 0))
    full_d = pl.BlockSpec((NP, 1), lambda c: (0, 0))

    gcn1 = pl.pallas_call(
        _gcn_first,
        grid=(NCHUNK,),
        in_specs=[espec_c, espec_r, full_h, full_w, full_b],
        out_specs=[full_h, full_d],
        out_shape=[jax.ShapeDtypeStruct((NP, F), f32),
                   jax.ShapeDtypeStruct((NP, 1), f32)],
        scratch_shapes=[pltpu.VMEM((NP, F), f32),
                        pltpu.VMEM((NP, F), f32),
                        pltpu.VMEM((NP, 1), f32)],
    )
    h1, deg = gcn1(ec, er, h0, W0, b0.reshape(1, F))

    gcn_n = pl.pallas_call(
        _gcn_next,
        grid=(NCHUNK,),
        in_specs=[espec_c, espec_r, full_h, full_w, full_b, full_d],
        out_specs=full_h,
        out_shape=jax.ShapeDtypeStruct((NP, F), f32),
        scratch_shapes=[pltpu.VMEM((NP, F), f32),
                        pltpu.VMEM((NP, F), f32)],
    )
    h2 = gcn_n(ec, er, h1, W1, b1.reshape(1, F), deg)
    h3 = gcn_n(ec, er, h2, W2, b2.reshape(1, F), deg)
    h4 = gcn_n(ec, er, h3, w3p, b3p.reshape(1, F), deg)

    key_row = h4[:, 0].reshape(1, NP)
    key_col = h4[:, 0:1]

    rank = pl.pallas_call(
        _rank_kernel,
        grid=(NP // RB,),
        in_specs=[pl.BlockSpec((RB, 1), lambda j: (j, 0)),
                  pl.BlockSpec((RB, 1), lambda j: (j, 0)),
                  pl.BlockSpec((1, NP), lambda j: (0, 0)),
                  pl.BlockSpec((1, NP), lambda j: (0, 0))],
        out_specs=pl.BlockSpec((1, NP), lambda j: (0, 0)),
        out_shape=jax.ShapeDtypeStruct((1, NP), f32),
        scratch_shapes=[pltpu.VMEM((1, NP), f32)],
    )(key_col, batch_col, key_row, batch_row)

    s_mat = pl.pallas_call(
        _select_kernel,
        grid=(NSLOT // SB,),
        in_specs=[pl.BlockSpec((1, NP), lambda b: (0, 0)),
                  pl.BlockSpec((1, NP), lambda b: (0, 0)),
                  pl.BlockSpec((NP, F), lambda b: (0, 0)),
                  pl.BlockSpec((NP, F), lambda b: (0, 0)),
                  pl.BlockSpec((NP, F), lambda b: (0, 0)),
                  pl.BlockSpec((NP, F), lambda b: (0, 0))],
        out_specs=pl.BlockSpec((SB, FP), lambda b: (b, 0)),
        out_shape=jax.ShapeDtypeStruct((NSLOT, FP), f32),
    )(rank, batch_row, h1, h2, h3, h4)

    wc1 = jnp.zeros((FP, 16), f32)
    wc1 = wc1.at[0:TL, :].set(conv1_w[:, 0, :].T)
    w2s = conv2_w.transpose(2, 1, 0)
    wr = d1_w.reshape(32, 32, TV).transpose(2, 1, 0)

    out = pl.pallas_call(
        _tail_kernel,
        in_specs=[pl.BlockSpec((NSLOT, FP), lambda: (0, 0)),
                  pl.BlockSpec((FP, 16), lambda: (0, 0)),
                  pl.BlockSpec((1, 16), lambda: (0, 0)),
                  pl.BlockSpec((5, 16, 32), lambda: (0, 0, 0)),
                  pl.BlockSpec((1, 32), lambda: (0, 0)),
                  pl.BlockSpec((TV, 32, 32), lambda: (0, 0, 0)),
                  pl.BlockSpec((1, 32), lambda: (0, 0)),
                  pl.BlockSpec((32, 10), lambda: (0, 0)),
                  pl.BlockSpec((1, 10), lambda: (0, 0))],
        out_specs=pl.BlockSpec((G, 10), lambda: (0, 0)),
        out_shape=jax.ShapeDtypeStruct((G, 10), f32),
    )(s_mat, wc1, conv1_b.reshape(1, 16), w2s, conv2_b.reshape(1, 32),
      wr, d1_b.reshape(1, 32), d2_w.T, d2_b.reshape(1, 10))
    return out


# speedup vs baseline: 1.0120x; 1.0120x over previous
import functools

import jax
import jax.numpy as jnp
from jax.experimental import pallas as pl
from jax.experimental.pallas import tpu as pltpu

N = 10000
NP = 10240
E = 320000
EC = 512
NCHUNK = E // EC
G = 64
K = 64
F = 128
TL = 3 * F + 1
FP = 512
NSLOT = G * K
SB = 256
RB = 256
T2 = 32
TV = 28


def _gcn_first(ec_ref, er_ref, h_ref, w_ref, b_ref, out_ref, deg_ref,
               hlin, acc, dacc):
    c = pl.program_id(0)

    @pl.when(c == 0)
    def _init():
        hlin[...] = jax.lax.dot_general(
            h_ref[...], w_ref[...], (((1,), (1,)), ((), ())),
            preferred_element_type=jnp.float32) + b_ref[...]
        acc[...] = jnp.zeros((NP, F), jnp.float32)
        dacc[...] = jnp.zeros((NP, 1), jnp.float32)

    src_col = ec_ref[0, :, 0:1]
    src_row = er_ref[0, 0:1, :]
    dst_row = er_ref[0, 1:2, :]
    lane_n = jax.lax.broadcasted_iota(jnp.int32, (EC, NP), 1)
    oh_src = (src_col == lane_n).astype(jnp.float32)
    g = jnp.dot(oh_src, hlin[...], preferred_element_type=jnp.float32)
    node_col = jax.lax.broadcasted_iota(jnp.int32, (NP, EC), 0)
    oh_dst_t = (node_col == dst_row).astype(jnp.float32)
    acc[...] += jnp.dot(oh_dst_t, g, preferred_element_type=jnp.float32)
    oh_src_t = (node_col == src_row).astype(jnp.float32)
    dacc[...] += jnp.sum(oh_src_t, axis=1, keepdims=True)

    @pl.when(c == NCHUNK - 1)
    def _fin():
        deg = dacc[...] + 1.0
        out_ref[...] = jnp.tanh((acc[...] + hlin[...]) / deg)
        deg_ref[...] = deg


def _gcn_next(ec_ref, er_ref, h_ref, w_ref, b_ref, deg_ref, out_ref,
              hlin, acc):
    c = pl.program_id(0)

    @pl.when(c == 0)
    def _init():
        hlin[...] = jax.lax.dot_general(
            h_ref[...], w_ref[...], (((1,), (1,)), ((), ())),
            preferred_element_type=jnp.float32) + b_ref[...]
        acc[...] = jnp.zeros((NP, F), jnp.float32)

    src_col = ec_ref[0, :, 0:1]
    dst_row = er_ref[0, 1:2, :]
    lane_n = jax.lax.broadcasted_iota(jnp.int32, (EC, NP), 1)
    oh_src = (src_col == lane_n).astype(jnp.float32)
    g = jnp.dot(oh_src, hlin[...], preferred_element_type=jnp.float32)
    node_col = jax.lax.broadcasted_iota(jnp.int32, (NP, EC), 0)
    oh_dst_t = (node_col == dst_row).astype(jnp.float32)
    acc[...] += jnp.dot(oh_dst_t, g, preferred_element_type=jnp.float32)

    @pl.when(c == NCHUNK - 1)
    def _fin():
        out_ref[...] = jnp.tanh((acc[...] + hlin[...]) / deg_ref[...])


def _rank_kernel(kc_ref, bc_ref, kr_ref, br_ref, rank_ref, racc):
    j = pl.program_id(0)

    @pl.when(j == 0)
    def _init():
        racc[...] = jnp.zeros((1, NP), jnp.float32)

    kc = kc_ref[...]
    bc = bc_ref[...]
    kr = kr_ref[...]
    br = br_ref[...]
    jidx = j * RB + jax.lax.broadcasted_iota(jnp.int32, (RB, 1), 0)
    iidx = jax.lax.broadcasted_iota(jnp.int32, (RB, NP), 1)
    beats = (kc > kr) | ((kc == kr) & (jidx < iidx))
    mask = (bc == br) & beats
    racc[...] += jnp.sum(mask.astype(jnp.float32), axis=0, keepdims=True)

    @pl.when(j == NP // RB - 1)
    def _fin():
        rank_ref[...] = racc[...]


def _select_kernel(rr_ref, br_ref, h1_ref, h2_ref, h3_ref, h4_ref, s_ref):
    b = pl.program_id(0)
    s = b * SB + jax.lax.broadcasted_iota(jnp.int32, (SB, 1), 0)
    par = jax.lax.shift_right_logical(s, 11)
    within = jax.lax.bitwise_and(s, 2047)
    gidx = jax.lax.shift_right_logical(within, 5)
    tidx = jax.lax.bitwise_and(within, 31)
    r = 2 * tidx + par
    oh = ((br_ref[...] == gidx) &
          (rr_ref[...] == r.astype(jnp.float32))).astype(jnp.float32)
    s_ref[:, 0:F] = jnp.dot(oh, h1_ref[...], preferred_element_type=jnp.float32)
    s_ref[:, F:2 * F] = jnp.dot(oh, h2_ref[...], preferred_element_type=jnp.float32)
    s_ref[:, 2 * F:3 * F] = jnp.dot(oh, h3_ref[...], preferred_element_type=jnp.float32)
    s_ref[:, 3 * F:4 * F] = jnp.dot(oh, h4_ref[...], preferred_element_type=jnp.float32)


def _tail_kernel(s_ref, wc1_ref, cb1_ref, w2_ref, cb2_ref, wr_ref, db1_ref,
                 w2t_ref, db2_ref, out_ref):
    s_even = s_ref[0:G * T2, :]
    s_odd = s_ref[G * T2:NSLOT, :]
    wc1 = wc1_ref[...]
    r1e = jnp.maximum(jnp.dot(s_even, wc1, preferred_element_type=jnp.float32)
                      + cb1_ref[...], 0.0)
    r1o = jnp.maximum(jnp.dot(s_odd, wc1, preferred_element_type=jnp.float32)
                      + cb1_ref[...], 0.0)
    pooled = jnp.maximum(r1e, r1o)
    c2 = jnp.zeros((G * T2, 32), jnp.float32)
    for dt in range(5):
        if dt == 0:
            sh = pooled
        else:
            sh = jnp.concatenate(
                [pooled[dt:, :], jnp.zeros((dt, 16), jnp.float32)], axis=0)
        c2 = c2 + jnp.dot(sh, w2_ref[dt, :, :],
                          preferred_element_type=jnp.float32)
    c2 = jnp.maximum(c2 + cb2_ref[...], 0.0)
    lane_gt = jax.lax.broadcasted_iota(jnp.int32, (G, G * T2), 1)
    grow = jax.lax.broadcasted_iota(jnp.int32, (G, G * T2), 0)
    racc = jnp.zeros((G, 32), jnp.float32)
    for t in range(TV):
        sel = (lane_gt == grow * T2 + t).astype(jnp.float32)
        ct = jnp.dot(sel, c2, preferred_element_type=jnp.float32)
        racc = racc + jnp.dot(ct, wr_ref[t, :, :],
                              preferred_element_type=jnp.float32)
    r1d = jnp.maximum(racc + db1_ref[...], 0.0)
    logits = jnp.dot(r1d, w2t_ref[...],
                     preferred_element_type=jnp.float32) + db2_ref[...]
    m = jnp.max(logits, axis=1, keepdims=True)
    e = jnp.exp(logits - m)
    out_ref[...] = e / jnp.sum(e, axis=1, keepdims=True)


def kernel(x, edge_index, batch, emb, W0, b0, W1, b1, W2, b2, W3, b3,
           conv1_w, conv1_b, conv2_w, conv2_b, d1_w, d1_b, d2_w, d2_b):
    f32 = jnp.float32
    h0 = emb[x].astype(f32)
    h0 = jnp.pad(h0, ((0, NP - N), (0, 0)))
    ec = edge_index.T.reshape(NCHUNK, EC, 2)
    er = edge_index.reshape(2, NCHUNK, EC).transpose(1, 0, 2)
    w3p = jnp.pad(W3, ((0, F - W3.shape[0]), (0, 0)))
    b3p = jnp.pad(b3, (0, F - b3.shape[0]))
    batch_p = jnp.pad(batch, (0, NP - N), constant_values=G)
    batch_row = batch_p.reshape(1, NP)
    batch_col = batch_p.reshape(NP, 1)

    espec_c = pl.BlockSpec((1, EC, 2), lambda c: (c, 0, 0))
    espec_r = pl.BlockSpec((1, 2, EC), lambda c: (c, 0, 0))
    full_h = pl.BlockSpec((NP, F), lambda c: (0, 0))
    full_w = pl.BlockSpec((F, F), lambda c: (0, 0))
    full_b = pl.BlockSpec((1, F), lambda c: (0, 0))
    full_d = pl.BlockSpec((NP, 1), lambda c: (0, 0))

    gcn1 = pl.pallas_call(
        _gcn_first,
        grid=(NCHUNK,),
        in_specs=[espec_c, espec_r, full_h, full_w, full_b],
        out_specs=[full_h, full_d],
        out_shape=[jax.ShapeDtypeStruct((NP, F), f32),
                   jax.ShapeDtypeStruct((NP, 1), f32)],
        scratch_shapes=[pltpu.VMEM((NP, F), f32),
                        pltpu.VMEM((NP, F), f32),
                        pltpu.VMEM((NP, 1), f32)],
    )
    h1, deg = gcn1(ec, er, h0, W0, b0.reshape(1, F))

    gcn_n = pl.pallas_call(
        _gcn_next,
        grid=(NCHUNK,),
        in_specs=[espec_c, espec_r, full_h, full_w, full_b, full_d],
        out_specs=full_h,
        out_shape=jax.ShapeDtypeStruct((NP, F), f32),
        scratch_shapes=[pltpu.VMEM((NP, F), f32),
                        pltpu.VMEM((NP, F), f32)],
    )
    h2 = gcn_n(ec, er, h1, W1, b1.reshape(1, F), deg)
    h3 = gcn_n(ec, er, h2, W2, b2.reshape(1, F), deg)
    h4 = gcn_n(ec, er, h3, w3p, b3p.reshape(1, F), deg)

    key_row = h4[:, 0].reshape(1, NP)
    key_col = h4[:, 0:1]

    rank = pl.pallas_call(
        _rank_kernel,
        grid=(NP // RB,),
        in_specs=[pl.BlockSpec((RB, 1), lambda j: (j, 0)),
                  pl.BlockSpec((RB, 1), lambda j: (j, 0)),
                  pl.BlockSpec((1, NP), lambda j: (0, 0)),
                  pl.BlockSpec((1, NP), lambda j: (0, 0))],
        out_specs=pl.BlockSpec((1, NP), lambda j: (0, 0)),
        out_shape=jax.ShapeDtypeStruct((1, NP), f32),
        scratch_shapes=[pltpu.VMEM((1, NP), f32)],
    )(key_col, batch_col, key_row, batch_row)

    s_mat = pl.pallas_call(
        _select_kernel,
        grid=(NSLOT // SB,),
        in_specs=[pl.BlockSpec((1, NP), lambda b: (0, 0)),
                  pl.BlockSpec((1, NP), lambda b: (0, 0)),
                  pl.BlockSpec((NP, F), lambda b: (0, 0)),
                  pl.BlockSpec((NP, F), lambda b: (0, 0)),
                  pl.BlockSpec((NP, F), lambda b: (0, 0)),
                  pl.BlockSpec((NP, F), lambda b: (0, 0))],
        out_specs=pl.BlockSpec((SB, FP), lambda b: (b, 0)),
        out_shape=jax.ShapeDtypeStruct((NSLOT, FP), f32),
    )(rank, batch_row, h1, h2, h3, h4)

    wc1 = jnp.zeros((FP, 16), f32)
    wc1 = wc1.at[0:TL, :].set(conv1_w[:, 0, :].T)
    w2s = conv2_w.transpose(2, 1, 0)
    wr = d1_w.reshape(32, 32, TV).transpose(2, 1, 0)

    out = pl.pallas_call(
        _tail_kernel,
        in_specs=[pl.BlockSpec((NSLOT, FP), lambda: (0, 0)),
                  pl.BlockSpec((FP, 16), lambda: (0, 0)),
                  pl.BlockSpec((1, 16), lambda: (0, 0)),
                  pl.BlockSpec((5, 16, 32), lambda: (0, 0, 0)),
                  pl.BlockSpec((1, 32), lambda: (0, 0)),
                  pl.BlockSpec((TV, 32, 32), lambda: (0, 0, 0)),
                  pl.BlockSpec((1, 32), lambda: (0, 0)),
                  pl.BlockSpec((32, 10), lambda: (0, 0)),
                  pl.BlockSpec((1, 10), lambda: (0, 0))],
        out_specs=pl.BlockSpec((G, 10), lambda: (0, 0)),
        out_shape=jax.ShapeDtypeStruct((G, 10), f32),
    )(s_mat, wc1, conv1_b.reshape(1, 16), w2s, conv2_b.reshape(1, 32),
      wr, d1_b.reshape(1, 32), d2_w.T, d2_b.reshape(1, 10))
    return out
